# Initial kernel scaffold; baseline (speedup 1.0000x reference)
import jax
import jax.numpy as jnp
from jax.experimental import pallas as pl


def kernel(x, edge_index, W_lin, b_lin, W_gate, b_gate):
    raise NotImplementedError("write your pallas kernel here")


# speedup vs baseline: 2.3551x; 2.3551x over previous
import functools

import jax
import jax.numpy as jnp
from jax import lax
from jax.experimental import pallas as pl
from jax.experimental.pallas import tpu as pltpu
from jax.experimental.pallas import tpu_sc as plsc

N = 10000
D = 128
DH = D // 2
E = 320000
NC = 2
NS = 16
BATCH = 128
EPT = 20480
NBATCH = EPT // BATCH
E_PAD = NS * EPT
PAD_IDX = N
TBL = N + 16
ACC = 10240
OUT_PER_TILE = ACC // NS


def _dense_tables(x, w1t, w2t, wlt, bg, bl):
    blk = 400

    def body(x_ref, w1_ref, w2_ref, wl_ref, bg_ref, bl_ref, a_ref, b_ref, c_ref):
        xb = x_ref[...]
        a_ref[...] = jnp.dot(xb, w1_ref[...], preferred_element_type=jnp.float32) + bg_ref[...]
        b_ref[...] = jnp.dot(xb, w2_ref[...], preferred_element_type=jnp.float32)
        c_ref[...] = jnp.dot(xb, wl_ref[...], preferred_element_type=jnp.float32) + bl_ref[...]

    return pl.pallas_call(
        body,
        grid=(N // blk,),
        in_specs=[
            pl.BlockSpec((blk, D), lambda i: (i, 0)),
            pl.BlockSpec((D, D), lambda i: (0, 0)),
            pl.BlockSpec((D, D), lambda i: (0, 0)),
            pl.BlockSpec((D, D), lambda i: (0, 0)),
            pl.BlockSpec((1, D), lambda i: (0, 0)),
            pl.BlockSpec((1, D), lambda i: (0, 0)),
        ],
        out_specs=[pl.BlockSpec((blk, D), lambda i: (i, 0))] * 3,
        out_shape=[jax.ShapeDtypeStruct((N, D), jnp.float32)] * 3,
    )(x, w1t, w2t, wlt, bg, bl)


@functools.partial(
    pl.kernel,
    out_type=jax.ShapeDtypeStruct((NC, ACC, DH), jnp.float32),
    mesh=plsc.VectorSubcoreMesh(core_axis_name="c", subcore_axis_name="s"),
    compiler_params=pltpu.CompilerParams(use_tc_tiling_on_sc=False),
    scratch_types=[
        pltpu.VMEM((BATCH,), jnp.int32),
        pltpu.VMEM((BATCH,), jnp.int32),
        pltpu.VMEM((BATCH, DH), jnp.float32),
        pltpu.VMEM((BATCH, DH), jnp.float32),
        pltpu.VMEM((BATCH, DH), jnp.float32),
        pltpu.VMEM((BATCH, DH), jnp.float32),
        pltpu.VMEM_SHARED((ACC, DH), jnp.float32),
        pltpu.SemaphoreType.DMA,
        pltpu.SemaphoreType.DMA,
        pltpu.SemaphoreType.DMA,
    ],
)
def _sc_edges(a_hbm, b_hbm, c_hbm, row_hbm, col_hbm, out_hbm,
              rowi_v, coli_v, a_v, b_v, c_v, msg_v, acc_sh,
              sem_a, sem_b, sem_c):
    cid = lax.axis_index("c")
    sid = lax.axis_index("s")
    base = sid * EPT

    def zrow(e, carry):
        for du in range(DH // 16):
            msg_v[e, pl.ds(du * 16, 16)] = jnp.zeros((16,), jnp.float32)
        return carry

    lax.fori_loop(0, BATCH, zrow, 0)
    for j in range(ACC // NS // BATCH):
        pltpu.sync_copy(msg_v, acc_sh.at[pl.ds(sid * (ACC // NS) + j * BATCH, BATCH)])
    plsc.subcore_barrier()

    def batch_body(jb, carry):
        eb = base + jb * BATCH
        pltpu.sync_copy(row_hbm.at[pl.ds(eb, BATCH)], rowi_v)
        pltpu.sync_copy(col_hbm.at[pl.ds(eb, BATCH)], coli_v)
        cp_a = pltpu.async_copy(a_hbm.at[cid].at[rowi_v], a_v, sem_a)
        cp_b = pltpu.async_copy(b_hbm.at[cid].at[coli_v], b_v, sem_b)
        cp_c = pltpu.async_copy(c_hbm.at[cid].at[coli_v], c_v, sem_c)
        cp_a.wait()
        cp_b.wait()
        cp_c.wait()

        def erow(e, inner):
            for du in range(DH // 16):
                sl = pl.ds(du * 16, 16)
                av = a_v[e, sl]
                bv = b_v[e, sl]
                cv = c_v[e, sl]
                msg_v[e, sl] = cv / (1.0 + jnp.exp(-(av + bv)))
            return inner

        lax.fori_loop(0, BATCH, erow, 0)
        pltpu.sync_copy(msg_v, acc_sh.at[rowi_v], add=True)
        return carry

    lax.fori_loop(0, NBATCH, batch_body, 0)

    plsc.subcore_barrier()
    pltpu.sync_copy(acc_sh.at[pl.ds(sid * OUT_PER_TILE, OUT_PER_TILE)],
                    out_hbm.at[cid, pl.ds(sid * OUT_PER_TILE, OUT_PER_TILE)])


def _final_add(partials, c_tbl):
    blk = 400

    def body(p_ref, c_ref, o_ref):
        o_ref[...] = jnp.concatenate([p_ref[0], p_ref[1]], axis=-1) + c_ref[...]

    return pl.pallas_call(
        body,
        grid=(N // blk,),
        in_specs=[
            pl.BlockSpec((NC, blk, DH), lambda i: (0, i, 0)),
            pl.BlockSpec((blk, D), lambda i: (i, 0)),
        ],
        out_specs=pl.BlockSpec((blk, D), lambda i: (i, 0)),
        out_shape=jax.ShapeDtypeStruct((N, D), jnp.float32),
    )(partials, c_tbl)


def kernel(x, edge_index, W_lin, b_lin, W_gate, b_gate):
    ei = edge_index.astype(jnp.int32)
    pad = jnp.full((E_PAD - E,), PAD_IDX, jnp.int32)
    row_p = jnp.concatenate([ei[0], pad])
    col_p = jnp.concatenate([ei[1], pad])

    w1t = W_gate[:, :D].T
    w2t = W_gate[:, D:].T
    wlt = W_lin.T
    a_tbl, b_tbl, c_tbl = _dense_tables(
        x, w1t, w2t, wlt, b_gate.reshape(1, D), b_lin.reshape(1, D))

    zpad = jnp.zeros((TBL - N, D), jnp.float32)

    def halves(t):
        tp = jnp.concatenate([t, zpad])
        return tp.reshape(TBL, NC, DH).transpose(1, 0, 2)

    partials = _sc_edges(halves(a_tbl), halves(b_tbl), halves(c_tbl),
                         row_p, col_p)

    return _final_add(partials, c_tbl)
